# baseline (speedup 1.0000x reference)
import functools

import jax
import jax.numpy as jnp
from jax import lax
from jax.experimental import pallas as pl
from jax.experimental.pallas import tpu as pltpu
from jax.experimental.pallas import tpu_sc as plsc

MAXLEN = 200
EMBED_DIM = 128
NC = 2
NS = 16
NW = NC * NS
LANES = 16
COLS = EMBED_DIM // LANES
RPD = 2


def _make_sc_kernel(batch):
    n = batch // NW
    groups = n // (2 * RPD)
    mesh = plsc.VectorSubcoreMesh(core_axis_name="c", subcore_axis_name="s")

    @functools.partial(
        pl.kernel,
        mesh=mesh,
        out_type=jax.ShapeDtypeStruct((batch, MAXLEN, EMBED_DIM), jnp.float32),
        compiler_params=pltpu.CompilerParams(use_tc_tiling_on_sc=True),
        scratch_types=[
            pltpu.VMEM((MAXLEN, EMBED_DIM), jnp.float32),
            pltpu.VMEM((RPD, MAXLEN, EMBED_DIM), jnp.float32),
            pltpu.VMEM((RPD, MAXLEN, EMBED_DIM), jnp.float32),
            pltpu.SemaphoreType.DMA,
            pltpu.SemaphoreType.DMA,
            pltpu.SemaphoreType.DMA,
            pltpu.SemaphoreType.DMA,
        ],
    )
    def sc_add(x_hbm, pos_hbm, out_hbm, pos_v, b0, b1,
               sin0, sin1, sout0, sout1):
        wid = lax.axis_index("s") * NC + lax.axis_index("c")
        base = wid * n
        pltpu.sync_copy(pos_hbm, pos_v)

        def in_copy(row, buf, sem):
            return pltpu.make_async_copy(x_hbm.at[pl.ds(row, RPD)], buf, sem)

        def out_copy(buf, row, sem):
            return pltpu.make_async_copy(buf, out_hbm.at[pl.ds(row, RPD)], sem)

        def compute(buf):
            for j in range(RPD):
                @plsc.parallel_loop(0, MAXLEN, step=1, unroll=4)
                def body(r):
                    for u in range(COLS):
                        buf[j, r, pl.ds(u * LANES, LANES)] = (
                            buf[j, r, pl.ds(u * LANES, LANES)]
                            + pos_v[r, pl.ds(u * LANES, LANES)]
                        )

        in_copy(base, b0, sin0).start()
        in_copy(base + RPD, b1, sin1).start()

        def main_body(k, c):
            r0 = base + 2 * RPD * k
            r1 = r0 + RPD
            in_copy(r0, b0, sin0).wait()
            compute(b0)
            out_copy(b0, r0, sout0).start()
            in_copy(r1, b1, sin1).wait()
            compute(b1)
            out_copy(b1, r1, sout1).start()
            out_copy(b0, r0, sout0).wait()
            in_copy(r0 + 2 * RPD, b0, sin0).start()
            out_copy(b1, r1, sout1).wait()
            in_copy(r1 + 2 * RPD, b1, sin1).start()
            return c

        lax.fori_loop(0, groups - 1, main_body, 0)

        r0 = base + 2 * RPD * (groups - 1)
        r1 = r0 + RPD
        in_copy(r0, b0, sin0).wait()
        compute(b0)
        out_copy(b0, r0, sout0).start()
        in_copy(r1, b1, sin1).wait()
        compute(b1)
        out_copy(b1, r1, sout1).start()
        out_copy(b0, r0, sout0).wait()
        out_copy(b1, r1, sout1).wait()

    return sc_add


def kernel(x, pos_table):
    return _make_sc_kernel(x.shape[0])(x, pos_table)


# speedup vs baseline: 1.1571x; 1.1571x over previous
import functools

import jax
import jax.numpy as jnp
from jax import lax
from jax.experimental import pallas as pl
from jax.experimental.pallas import tpu as pltpu
from jax.experimental.pallas import tpu_sc as plsc

MAXLEN = 200
EMBED_DIM = 128
NC = 2
NS = 16
NW = NC * NS
LANES = 16
COLS = EMBED_DIM // LANES
HA = 104
HB = MAXLEN - HA


def _make_sc_kernel(batch):
    n = batch // NW
    mesh = plsc.VectorSubcoreMesh(core_axis_name="c", subcore_axis_name="s")

    @functools.partial(
        pl.kernel,
        mesh=mesh,
        out_type=jax.ShapeDtypeStruct((batch, MAXLEN, EMBED_DIM), jnp.float32),
        compiler_params=pltpu.CompilerParams(use_tc_tiling_on_sc=True),
        scratch_types=[
            pltpu.VMEM((MAXLEN, EMBED_DIM), jnp.float32),
            pltpu.VMEM((HA, EMBED_DIM), jnp.float32),
            pltpu.VMEM((HA, EMBED_DIM), jnp.float32),
            pltpu.VMEM((HB, EMBED_DIM), jnp.float32),
            pltpu.VMEM((HB, EMBED_DIM), jnp.float32),
            pltpu.VMEM((HA, EMBED_DIM), jnp.float32),
            pltpu.VMEM((HA, EMBED_DIM), jnp.float32),
            pltpu.VMEM((HB, EMBED_DIM), jnp.float32),
            pltpu.VMEM((HB, EMBED_DIM), jnp.float32),
            pltpu.SemaphoreType.DMA,
            pltpu.SemaphoreType.DMA,
            pltpu.SemaphoreType.DMA,
            pltpu.SemaphoreType.DMA,
            pltpu.SemaphoreType.DMA,
            pltpu.SemaphoreType.DMA,
            pltpu.SemaphoreType.DMA,
            pltpu.SemaphoreType.DMA,
        ],
    )
    def sc_add(x_hbm, pos_hbm, out_hbm, pos_v,
               xa0, xa1, xb0, xb1, oa0, oa1, ob0, ob1,
               sia0, sia1, sib0, sib1, soa0, soa1, sob0, sob1):
        wid = lax.axis_index("s") * NC + lax.axis_index("c")
        base = wid * n
        pltpu.sync_copy(pos_hbm, pos_v)

        def ina(row, buf, sem):
            return pltpu.make_async_copy(
                x_hbm.at[row, pl.ds(0, HA)], buf, sem)

        def inb(row, buf, sem):
            return pltpu.make_async_copy(
                x_hbm.at[row, pl.ds(HA, HB)], buf, sem)

        def outa(buf, row, sem):
            return pltpu.make_async_copy(
                buf, out_hbm.at[row, pl.ds(0, HA)], sem)

        def outb(buf, row, sem):
            return pltpu.make_async_copy(
                buf, out_hbm.at[row, pl.ds(HA, HB)], sem)

        def compute(src, dst, p0, rows):
            @plsc.parallel_loop(0, rows, step=1, unroll=4)
            def body(r):
                for u in range(COLS):
                    dst[r, pl.ds(u * LANES, LANES)] = (
                        src[r, pl.ds(u * LANES, LANES)]
                        + pos_v[p0 + r, pl.ds(u * LANES, LANES)]
                    )

        ina(base + 0, xa0, sia0).start()
        inb(base + 0, xb0, sib0).start()
        ina(base + 1, xa1, sia1).start()
        inb(base + 1, xb1, sib1).start()

        ina(base + 0, xa0, sia0).wait()
        compute(xa0, oa0, 0, HA)
        outa(oa0, base + 0, soa0).start()
        ina(base + 2, xa0, sia0).start()
        inb(base + 0, xb0, sib0).wait()
        compute(xb0, ob0, HA, HB)
        outb(ob0, base + 0, sob0).start()
        inb(base + 2, xb0, sib0).start()

        ina(base + 1, xa1, sia1).wait()
        compute(xa1, oa1, 0, HA)
        outa(oa1, base + 1, soa1).start()
        ina(base + 3, xa1, sia1).start()
        inb(base + 1, xb1, sib1).wait()
        compute(xb1, ob1, HA, HB)
        outb(ob1, base + 1, sob1).start()
        inb(base + 3, xb1, sib1).start()

        def main_body(k, c):
            re = base + 2 + 2 * k
            ina(re, xa0, sia0).wait()
            outa(oa0, re, soa0).wait()
            compute(xa0, oa0, 0, HA)
            outa(oa0, re, soa0).start()
            ina(re + 2, xa0, sia0).start()
            inb(re, xb0, sib0).wait()
            outb(ob0, re, sob0).wait()
            compute(xb0, ob0, HA, HB)
            outb(ob0, re, sob0).start()
            inb(re + 2, xb0, sib0).start()

            ro = re + 1
            ina(ro, xa1, sia1).wait()
            outa(oa1, ro, soa1).wait()
            compute(xa1, oa1, 0, HA)
            outa(oa1, ro, soa1).start()
            ina(ro + 2, xa1, sia1).start()
            inb(ro, xb1, sib1).wait()
            outb(ob1, ro, sob1).wait()
            compute(xb1, ob1, HA, HB)
            outb(ob1, ro, sob1).start()
            inb(ro + 2, xb1, sib1).start()
            return c

        lax.fori_loop(0, (n - 4) // 2, main_body, 0)

        re = base + n - 2
        ina(re, xa0, sia0).wait()
        outa(oa0, re, soa0).wait()
        compute(xa0, oa0, 0, HA)
        outa(oa0, re, soa0).start()
        inb(re, xb0, sib0).wait()
        outb(ob0, re, sob0).wait()
        compute(xb0, ob0, HA, HB)
        outb(ob0, re, sob0).start()

        ro = base + n - 1
        ina(ro, xa1, sia1).wait()
        outa(oa1, ro, soa1).wait()
        compute(xa1, oa1, 0, HA)
        outa(oa1, ro, soa1).start()
        inb(ro, xb1, sib1).wait()
        outb(ob1, ro, sob1).wait()
        compute(xb1, ob1, HA, HB)
        outb(ob1, ro, sob1).start()

        outa(oa0, re, soa0).wait()
        outb(ob0, re, sob0).wait()
        outa(oa1, ro, soa1).wait()
        outb(ob1, ro, sob1).wait()

    return sc_add


def kernel(x, pos_table):
    return _make_sc_kernel(x.shape[0])(x, pos_table)


# speedup vs baseline: 1.1604x; 1.0029x over previous
import functools

import jax
import jax.numpy as jnp
from jax import lax
from jax.experimental import pallas as pl
from jax.experimental.pallas import tpu as pltpu
from jax.experimental.pallas import tpu_sc as plsc

MAXLEN = 200
EMBED_DIM = 128
NC = 2
NS = 16
NW = NC * NS
LANES = 16
COLS = EMBED_DIM // LANES


def _make_sc_kernel(batch):
    n = batch // NW
    mesh = plsc.VectorSubcoreMesh(core_axis_name="c", subcore_axis_name="s")

    @functools.partial(
        pl.kernel,
        mesh=mesh,
        out_type=jax.ShapeDtypeStruct((batch, MAXLEN, EMBED_DIM), jnp.float32),
        compiler_params=pltpu.CompilerParams(use_tc_tiling_on_sc=True),
        scratch_types=[
            pltpu.VMEM((MAXLEN, EMBED_DIM), jnp.float32),
            pltpu.VMEM((MAXLEN, EMBED_DIM), jnp.float32),
            pltpu.VMEM((MAXLEN, EMBED_DIM), jnp.float32),
            pltpu.VMEM((MAXLEN, EMBED_DIM), jnp.float32),
            pltpu.VMEM((MAXLEN, EMBED_DIM), jnp.float32),
            pltpu.SemaphoreType.DMA,
            pltpu.SemaphoreType.DMA,
            pltpu.SemaphoreType.DMA,
            pltpu.SemaphoreType.DMA,
        ],
    )
    def sc_add(x_hbm, pos_hbm, out_hbm, pos_v, xb0, xb1, ob0, ob1,
               sin0, sin1, sout0, sout1):
        wid = lax.axis_index("s") * NC + lax.axis_index("c")
        base = wid * n
        pltpu.sync_copy(pos_hbm, pos_v)

        def in_copy(row, buf, sem):
            return pltpu.make_async_copy(x_hbm.at[row], buf, sem)

        def out_copy(buf, row, sem):
            return pltpu.make_async_copy(buf, out_hbm.at[row], sem)

        def compute(src, dst):
            @plsc.parallel_loop(0, MAXLEN, step=1, unroll=4)
            def body(r):
                for u in range(COLS):
                    dst[r, pl.ds(u * LANES, LANES)] = (
                        src[r, pl.ds(u * LANES, LANES)]
                        + pos_v[r, pl.ds(u * LANES, LANES)]
                    )

        in_copy(base + 0, xb0, sin0).start()
        in_copy(base + 1, xb1, sin1).start()

        in_copy(base + 0, xb0, sin0).wait()
        compute(xb0, ob0)
        out_copy(ob0, base + 0, sout0).start()
        in_copy(base + 2, xb0, sin0).start()

        in_copy(base + 1, xb1, sin1).wait()
        compute(xb1, ob1)
        out_copy(ob1, base + 1, sout1).start()
        in_copy(base + 3, xb1, sin1).start()

        def main_body(k, c):
            re = base + 2 + 2 * k
            in_copy(re, xb0, sin0).wait()
            out_copy(ob0, re, sout0).wait()
            compute(xb0, ob0)
            out_copy(ob0, re, sout0).start()
            in_copy(re + 2, xb0, sin0).start()

            ro = re + 1
            in_copy(ro, xb1, sin1).wait()
            out_copy(ob1, ro, sout1).wait()
            compute(xb1, ob1)
            out_copy(ob1, ro, sout1).start()
            in_copy(ro + 2, xb1, sin1).start()
            return c

        lax.fori_loop(0, (n - 4) // 2, main_body, 0)

        re = base + n - 2
        in_copy(re, xb0, sin0).wait()
        out_copy(ob0, re, sout0).wait()
        compute(xb0, ob0)
        out_copy(ob0, re, sout0).start()

        ro = base + n - 1
        in_copy(ro, xb1, sin1).wait()
        out_copy(ob1, ro, sout1).wait()
        compute(xb1, ob1)
        out_copy(ob1, ro, sout1).start()

        out_copy(ob0, re, sout0).wait()
        out_copy(ob1, ro, sout1).wait()

    return sc_add


def kernel(x, pos_table):
    return _make_sc_kernel(x.shape[0])(x, pos_table)
